# Initial kernel scaffold; baseline (speedup 1.0000x reference)
import jax
import jax.numpy as jnp
from jax.experimental import pallas as pl


def kernel(x, adj, AX_residual, weight, bias):
    raise NotImplementedError("write your pallas kernel here")


# speedup vs baseline: 1.0008x; 1.0008x over previous
import jax
import jax.numpy as jnp
from jax.experimental import pallas as pl
from jax.experimental.pallas import tpu as pltpu

_BM = 200


def _make_body(bm):
    def _gcn_body(x_ref, adj_ref, r_ref, w_ref, b_ref, out_ref, msg_ref):
        i = pl.program_id(0)
        acc = jnp.dot(adj_ref[...], x_ref[...], preferred_element_type=jnp.float32)
        ax = (acc + r_ref[...]) * 0.5
        xi = x_ref[pl.ds(i * bm, bm), :]
        msg = jnp.dot(ax, w_ref[...], preferred_element_type=jnp.float32)
        ixw = jnp.dot(xi, w_ref[...], preferred_element_type=jnp.float32)
        msg_ref[...] = msg
        out_ref[...] = msg + ixw + b_ref[...]

    return _gcn_body


def kernel(x, adj, AX_residual, weight, bias):
    n, d = x.shape
    bm = _BM if n % _BM == 0 else 8
    grid = (n // bm,)
    out_shape = [
        jax.ShapeDtypeStruct((n, d), jnp.float32),
        jax.ShapeDtypeStruct((n, d), jnp.float32),
    ]
    out, msg = pl.pallas_call(
        _make_body(bm),
        grid=grid,
        in_specs=[
            pl.BlockSpec(memory_space=pltpu.VMEM),
            pl.BlockSpec((bm, n), lambda i: (i, 0)),
            pl.BlockSpec((bm, d), lambda i: (i, 0)),
            pl.BlockSpec(memory_space=pltpu.VMEM),
            pl.BlockSpec(memory_space=pltpu.VMEM),
        ],
        out_specs=[
            pl.BlockSpec((bm, d), lambda i: (i, 0)),
            pl.BlockSpec((bm, d), lambda i: (i, 0)),
        ],
        out_shape=out_shape,
        compiler_params=pltpu.CompilerParams(
            dimension_semantics=("arbitrary",),
            vmem_limit_bytes=100 * 1024 * 1024,
        ),
    )(x, adj, AX_residual, weight, bias.reshape(1, d))
    return out, msg
